# Initial kernel scaffold; baseline (speedup 1.0000x reference)
import jax
import jax.numpy as jnp
from jax.experimental import pallas as pl


def kernel(sequence_tokens, structure_tokens, average_plddt, per_res_plddt, ss8_tokens, sasa_tokens, function_tokens, residue_annotation_tokens, seq_table, plddt_W, plddt_b, struc_plddt_W, struc_plddt_b, structure_table, ss8_table, sasa_table, func_tables, residue_table):
    raise NotImplementedError("write your pallas kernel here")


# speedup vs baseline: 1.3214x; 1.3214x over previous
import functools

import jax
import jax.numpy as jnp
from jax import lax
from jax.experimental import pallas as pl
from jax.experimental.pallas import tpu as pltpu
from jax.experimental.pallas import tpu_sc as plsc

D = 1536
L = 2048
FSEG = 192
NW = 32
ROWS_PER_W = L // NW
N_CONTRIB = 20
BLK = 8
N_IT = N_CONTRIB * (ROWS_PER_W // BLK)


def _tc_base_body(prp_ref, avg_ref, cnt0_ref, ftok_ref, pw_ref, pb_ref,
                  sw_ref, sb_ref, r0_ref, ft_ref, out_ref):
    f32 = jnp.float32
    centers = (lax.broadcasted_iota(jnp.int32, (1, 16), 1)
               .astype(f32) * (1.0 / 15.0))
    p = prp_ref[0, 0, :]
    z = (p[:, None] - centers) * 16.0
    e = jnp.exp(-(z * z))
    spr = lax.dot_general(e, sw_ref[...], (((1,), (1,)), ((), ())),
                          preferred_element_type=f32)
    a = avg_ref[0, 0]
    za = (a - centers) * 16.0
    ea = jnp.exp(-(za * za))
    pe = lax.dot_general(ea, pw_ref[...], (((1,), (1,)), ((), ())),
                         preferred_element_type=f32)
    ftok = ftok_ref[0]
    iota = lax.broadcasted_iota(jnp.int32, (256, 260), 1)
    parts = []
    for i in range(8):
        oh = jnp.where((ftok[:, i][:, None] == iota) & (iota > 0),
                       f32(1.0), f32(0.0))
        parts.append(lax.dot_general(oh, ft_ref[i], (((1,), (0,)), ((), ())),
                                     preferred_element_type=f32))
    femb = jnp.concatenate(parts, axis=1)
    corr = cnt0_ref[0, 0, :][:, None] * r0_ref[...]
    out_ref[...] = spr + pe + pb_ref[...] + sb_ref[...] + femb - corr


def _sc_body(base_ref, comb_ref, idxw_ref, out_ref, acc, stage, idxv, sem):
    wid = lax.axis_index("s") * 2 + lax.axis_index("c")
    row0 = wid * ROWS_PER_W
    pltpu.sync_copy(idxw_ref.at[wid], idxv)
    pltpu.sync_copy(base_ref.at[pl.ds(row0, ROWS_PER_W)], acc)

    def issue(it):
        buf = lax.rem(it, 2)
        pltpu.async_copy(comb_ref.at[idxv.at[pl.ds(it * BLK, BLK)]],
                         stage.at[buf], sem)

    issue(0)

    def step(it, _):
        pltpu.make_async_copy(comb_ref.at[pl.ds(0, BLK)], stage.at[0],
                              sem).wait()

        @pl.when(it < N_IT - 1)
        def _():
            issue(it + 1)

        buf = lax.rem(it, 2)
        roff = lax.rem(it, ROWS_PER_W // BLK) * BLK
        for r in range(BLK):
            for g in range(0, D // 16, 8):
                vs = [stage[buf, r, pl.ds((g + k) * 16, 16)]
                      for k in range(8)]
                for k in range(8):
                    plsc.addupdate(acc.at[roff + r, pl.ds((g + k) * 16, 16)],
                                   vs[k])
        return 0

    lax.fori_loop(0, N_IT, step, 0, unroll=False)
    pltpu.sync_copy(acc, out_ref.at[pl.ds(row0, ROWS_PER_W)])


def kernel(sequence_tokens, structure_tokens, average_plddt, per_res_plddt,
           ss8_tokens, sasa_tokens, function_tokens,
           residue_annotation_tokens, seq_table, plddt_W, plddt_b,
           struc_plddt_W, struc_plddt_b, structure_table, ss8_table,
           sasa_table, func_tables, residue_table):
    f32 = jnp.float32
    i32 = jnp.int32
    rtok = residue_annotation_tokens.astype(i32)
    count0 = jnp.sum((rtok == 0).astype(f32), axis=1)

    base = pl.pallas_call(
        _tc_base_body,
        grid=(8,),
        in_specs=[
            pl.BlockSpec((1, 1, 256), lambda b: (b, 0, 0)),
            pl.BlockSpec((1, 1), lambda b: (0, 0)),
            pl.BlockSpec((1, 1, 256), lambda b: (b, 0, 0)),
            pl.BlockSpec((1, 256, 8), lambda b: (b, 0, 0)),
            pl.BlockSpec((D, 16), lambda b: (0, 0)),
            pl.BlockSpec((1, D), lambda b: (0, 0)),
            pl.BlockSpec((D, 16), lambda b: (0, 0)),
            pl.BlockSpec((1, D), lambda b: (0, 0)),
            pl.BlockSpec((1, D), lambda b: (0, 0)),
            pl.BlockSpec((8, 260, FSEG), lambda b: (0, 0, 0)),
        ],
        out_specs=pl.BlockSpec((256, D), lambda b: (b, 0)),
        out_shape=jax.ShapeDtypeStruct((L, D), f32),
    )(per_res_plddt.reshape(8, 1, 256).astype(f32),
      average_plddt.reshape(1, 1).astype(f32),
      count0.reshape(8, 1, 256),
      function_tokens.astype(i32).reshape(8, 256, 8),
      plddt_W, plddt_b.reshape(1, D),
      struc_plddt_W, struc_plddt_b.reshape(1, D),
      residue_table[0:1, :], func_tables)

    comb = jnp.concatenate([seq_table, structure_table, ss8_table,
                            sasa_table, residue_table], axis=0)
    offs = [0, 64, 64 + 4101, 64 + 4101 + 11, 64 + 4101 + 11 + 19]
    idx_all = jnp.stack([sequence_tokens.astype(i32) + offs[0],
                         structure_tokens.astype(i32) + offs[1],
                         ss8_tokens.astype(i32) + offs[2],
                         sasa_tokens.astype(i32) + offs[3]]
                        + [rtok[:, s] + offs[4] for s in range(16)])
    idxw = (idx_all.reshape(N_CONTRIB, NW, ROWS_PER_W)
            .transpose(1, 0, 2)
            .reshape(NW, N_CONTRIB * ROWS_PER_W))

    mesh = plsc.VectorSubcoreMesh(core_axis_name="c", subcore_axis_name="s",
                                  num_cores=2, num_subcores=16)
    sc = functools.partial(
        pl.kernel,
        out_type=jax.ShapeDtypeStruct((L, D), f32),
        mesh=mesh,
        scratch_types=[
            pltpu.VMEM((ROWS_PER_W, D), f32),
            pltpu.VMEM((2, BLK, D), f32),
            pltpu.VMEM((N_CONTRIB * ROWS_PER_W,), i32),
            pltpu.SemaphoreType.DMA,
        ],
    )(_sc_body)
    return sc(base, comb, idxw)
